# Initial kernel scaffold; baseline (speedup 1.0000x reference)
import jax
import jax.numpy as jnp
from jax.experimental import pallas as pl

_R = 256
_C = 256
_THR = 0.5
_EPS = 1e-9


def _nms_kernel(x1_ref, y1_ref, x2_ref, y2_ref, ar_ref, out_ref):
    t = pl.program_id(0)
    base = t * _R

    @pl.when(t == 0)
    def _init():
        out_ref[...] = jnp.ones_like(out_ref)

    xr1 = x1_ref[:, pl.ds(base, _R)].reshape(_R, 1)
    yr1 = y1_ref[:, pl.ds(base, _R)].reshape(_R, 1)
    xr2 = x2_ref[:, pl.ds(base, _R)].reshape(_R, 1)
    yr2 = y2_ref[:, pl.ds(base, _R)].reshape(_R, 1)
    arr = ar_ref[:, pl.ds(base, _R)].reshape(_R, 1)

    def cross_body(c, sup):
        cb = c * _C
        xc1 = x1_ref[:, pl.ds(cb, _C)]
        yc1 = y1_ref[:, pl.ds(cb, _C)]
        xc2 = x2_ref[:, pl.ds(cb, _C)]
        yc2 = y2_ref[:, pl.ds(cb, _C)]
        arc = ar_ref[:, pl.ds(cb, _C)]
        kept = out_ref[:, pl.ds(cb, _C)] > 0.0
        w = jnp.maximum(jnp.minimum(xr2, xc2) - jnp.maximum(xr1, xc1), 0.0)
        h = jnp.maximum(jnp.minimum(yr2, yc2) - jnp.maximum(yr1, yc1), 0.0)
        inter = w * h
        iou = inter / (arr + arc - inter + _EPS)
        hit = (iou > _THR) & kept
        return sup | jnp.any(hit, axis=1, keepdims=True)

    sup = jax.lax.fori_loop(
        0, t, cross_body, jnp.zeros((_R, 1), jnp.bool_), unroll=False
    )
    out_ref[:, pl.ds(base, _R)] = jnp.where(sup.reshape(1, _R), 0.0, 1.0)

    tx1 = x1_ref[:, pl.ds(base, _R)]
    ty1 = y1_ref[:, pl.ds(base, _R)]
    tx2 = x2_ref[:, pl.ds(base, _R)]
    ty2 = y2_ref[:, pl.ds(base, _R)]
    tar = ar_ref[:, pl.ds(base, _R)]
    j = jax.lax.broadcasted_iota(jnp.int32, (1, _R), 1)

    def intra_body(k, carry):
        idx = base + k
        kx1 = x1_ref[:, pl.ds(idx, 1)]
        ky1 = y1_ref[:, pl.ds(idx, 1)]
        kx2 = x2_ref[:, pl.ds(idx, 1)]
        ky2 = y2_ref[:, pl.ds(idx, 1)]
        kar = ar_ref[:, pl.ds(idx, 1)]
        kk = out_ref[:, pl.ds(idx, 1)] > 0.0
        w = jnp.maximum(jnp.minimum(kx2, tx2) - jnp.maximum(kx1, tx1), 0.0)
        h = jnp.maximum(jnp.minimum(ky2, ty2) - jnp.maximum(ky1, ty1), 0.0)
        inter = w * h
        iou = inter / (tar + kar - inter + _EPS)
        suppress = (iou > _THR) & (j > k) & kk
        cur = out_ref[:, pl.ds(base, _R)]
        out_ref[:, pl.ds(base, _R)] = jnp.where(suppress, 0.0, cur)
        return carry

    jax.lax.fori_loop(0, _R, intra_body, 0, unroll=False)


def _run_nms(x1, y1, x2, y2, area, interpret=False):
    npad = x1.shape[1]
    spec = pl.BlockSpec((1, npad), lambda t: (0, 0))
    keep = pl.pallas_call(
        _nms_kernel,
        grid=(npad // _R,),
        in_specs=[spec] * 5,
        out_specs=spec,
        out_shape=jax.ShapeDtypeStruct((1, npad), jnp.float32),
        interpret=interpret,
    )(x1, y1, x2, y2, area)
    return keep[0]


def kernel(boxes, scores):
    n = scores.shape[0]
    order = jnp.argsort(-scores)
    b = boxes[order]
    npad = ((n + _R - 1) // _R) * _R
    pad = npad - n
    bp = jnp.concatenate([b, jnp.full((pad, 4), 2e9, jnp.float32)], axis=0)
    x1 = bp[:, 0].reshape(1, npad)
    y1 = bp[:, 1].reshape(1, npad)
    x2 = bp[:, 2].reshape(1, npad)
    y2 = bp[:, 3].reshape(1, npad)
    area = (x2 - x1) * (y2 - y1)
    keep_sorted = _run_nms(x1, y1, x2, y2, area)[:n] > 0.0
    kept = jnp.zeros((n,), dtype=bool).at[order].set(keep_sorted)
    return jnp.where(kept, scores, 0.0)


# speedup vs baseline: 40.7557x; 40.7557x over previous
import jax
import jax.numpy as jnp
from jax.experimental import pallas as pl

_R = 256
_C = 256
_THR = 0.5
_EPS = 1e-9


def _nms_kernel(x1_ref, y1_ref, x2_ref, y2_ref, ar_ref, out_ref):
    t = pl.program_id(0)
    base = t * _R

    @pl.when(t == 0)
    def _init():
        out_ref[...] = jnp.ones_like(out_ref)

    xr1 = x1_ref[:, pl.ds(base, _R)].reshape(_R, 1)
    yr1 = y1_ref[:, pl.ds(base, _R)].reshape(_R, 1)
    xr2 = x2_ref[:, pl.ds(base, _R)].reshape(_R, 1)
    yr2 = y2_ref[:, pl.ds(base, _R)].reshape(_R, 1)
    arr = ar_ref[:, pl.ds(base, _R)].reshape(_R, 1)

    def cross_body(c, sup):
        cb = c * _C
        xc1 = x1_ref[:, pl.ds(cb, _C)]
        yc1 = y1_ref[:, pl.ds(cb, _C)]
        xc2 = x2_ref[:, pl.ds(cb, _C)]
        yc2 = y2_ref[:, pl.ds(cb, _C)]
        arc = ar_ref[:, pl.ds(cb, _C)]
        kept = out_ref[:, pl.ds(cb, _C)] > 0.0
        w = jnp.maximum(jnp.minimum(xr2, xc2) - jnp.maximum(xr1, xc1), 0.0)
        h = jnp.maximum(jnp.minimum(yr2, yc2) - jnp.maximum(yr1, yc1), 0.0)
        inter = w * h
        iou = inter / (arr + arc - inter + _EPS)
        hit = jnp.where((iou > _THR) & kept, 1.0, 0.0)
        hit_row = jnp.max(hit, axis=1, keepdims=True).reshape(1, _R)
        return jnp.maximum(sup, hit_row)

    sup = jax.lax.fori_loop(
        0, t, cross_body, jnp.zeros((1, _R), jnp.float32), unroll=False
    )
    out_ref[:, pl.ds(base, _R)] = jnp.where(sup > 0.0, 0.0, 1.0)

    tx1 = x1_ref[:, pl.ds(base, _R)]
    ty1 = y1_ref[:, pl.ds(base, _R)]
    tx2 = x2_ref[:, pl.ds(base, _R)]
    ty2 = y2_ref[:, pl.ds(base, _R)]
    tar = ar_ref[:, pl.ds(base, _R)]
    j = jax.lax.broadcasted_iota(jnp.int32, (1, _R), 1)

    def intra_body(k, carry):
        m = j == k
        kx1 = jnp.sum(jnp.where(m, tx1, 0.0), axis=1, keepdims=True)
        ky1 = jnp.sum(jnp.where(m, ty1, 0.0), axis=1, keepdims=True)
        kx2 = jnp.sum(jnp.where(m, tx2, 0.0), axis=1, keepdims=True)
        ky2 = jnp.sum(jnp.where(m, ty2, 0.0), axis=1, keepdims=True)
        kar = jnp.sum(jnp.where(m, tar, 0.0), axis=1, keepdims=True)
        cur = out_ref[:, pl.ds(base, _R)]
        kk = jnp.max(jnp.where(m, cur, 0.0), axis=1, keepdims=True) > 0.0
        w = jnp.maximum(jnp.minimum(kx2, tx2) - jnp.maximum(kx1, tx1), 0.0)
        h = jnp.maximum(jnp.minimum(ky2, ty2) - jnp.maximum(ky1, ty1), 0.0)
        inter = w * h
        iou = inter / (tar + kar - inter + _EPS)
        suppress = (iou > _THR) & (j > k) & kk
        out_ref[:, pl.ds(base, _R)] = jnp.where(suppress, 0.0, cur)
        return carry

    jax.lax.fori_loop(0, _R, intra_body, 0, unroll=False)


def _run_nms(x1, y1, x2, y2, area, interpret=False):
    npad = x1.shape[1]
    spec = pl.BlockSpec((1, npad), lambda t: (0, 0))
    keep = pl.pallas_call(
        _nms_kernel,
        grid=(npad // _R,),
        in_specs=[spec] * 5,
        out_specs=spec,
        out_shape=jax.ShapeDtypeStruct((1, npad), jnp.float32),
        interpret=interpret,
    )(x1, y1, x2, y2, area)
    return keep[0]


def kernel(boxes, scores):
    n = scores.shape[0]
    order = jnp.argsort(-scores)
    b = boxes[order]
    npad = ((n + _R - 1) // _R) * _R
    pad = npad - n
    bp = jnp.concatenate([b, jnp.full((pad, 4), 2e9, jnp.float32)], axis=0)
    x1 = bp[:, 0].reshape(1, npad)
    y1 = bp[:, 1].reshape(1, npad)
    x2 = bp[:, 2].reshape(1, npad)
    y2 = bp[:, 3].reshape(1, npad)
    area = (x2 - x1) * (y2 - y1)
    keep_sorted = _run_nms(x1, y1, x2, y2, area)[:n] > 0.0
    kept = jnp.zeros((n,), dtype=bool).at[order].set(keep_sorted)
    return jnp.where(kept, scores, 0.0)


# speedup vs baseline: 54.3453x; 1.3334x over previous
import functools

import jax
import jax.numpy as jnp
from jax.experimental import pallas as pl
from jax.experimental.pallas import tpu as pltpu

_R = 256
_C = 1024
_S = 8
_THR = 0.5
_EPS = 1e-9
_BIG = 2.0e9


def _nms_kernel(cchunk, x1_ref, y1_ref, x2_ref, y2_ref, ar_ref, out_ref,
                px1_ref, py1_ref, px2_ref, py2_ref, par_ref,
                sx1_ref, sy1_ref, sx2_ref, sy2_ref, sar_ref):
    t = pl.program_id(0)
    base = t * _R

    @pl.when(t == 0)
    def _init():
        px1_ref[...] = jnp.full_like(px1_ref, _BIG)
        py1_ref[...] = jnp.full_like(py1_ref, _BIG)
        px2_ref[...] = jnp.full_like(px2_ref, _BIG)
        py2_ref[...] = jnp.full_like(py2_ref, _BIG)
        par_ref[...] = jnp.zeros_like(par_ref)

    xr1 = x1_ref[:, pl.ds(base, _R)].reshape(_R, 1)
    yr1 = y1_ref[:, pl.ds(base, _R)].reshape(_R, 1)
    xr2 = x2_ref[:, pl.ds(base, _R)].reshape(_R, 1)
    yr2 = y2_ref[:, pl.ds(base, _R)].reshape(_R, 1)
    arr = ar_ref[:, pl.ds(base, _R)].reshape(_R, 1)
    sx1_ref[...] = xr1
    sy1_ref[...] = yr1
    sx2_ref[...] = xr2
    sy2_ref[...] = yr2
    sar_ref[...] = arr

    def cross_body(c, sup):
        cb = c * cchunk
        xc1 = px1_ref[:, pl.ds(cb, cchunk)]
        yc1 = py1_ref[:, pl.ds(cb, cchunk)]
        xc2 = px2_ref[:, pl.ds(cb, cchunk)]
        yc2 = py2_ref[:, pl.ds(cb, cchunk)]
        arc = par_ref[:, pl.ds(cb, cchunk)]
        w = jnp.maximum(jnp.minimum(xr2, xc2) - jnp.maximum(xr1, xc1), 0.0)
        h = jnp.maximum(jnp.minimum(yr2, yc2) - jnp.maximum(yr1, yc1), 0.0)
        inter = w * h
        iou = inter / (arc + arr - inter + _EPS)
        row_max = jnp.max(iou, axis=1, keepdims=True).reshape(1, _R)
        return jnp.maximum(sup, row_max)

    nchunks = (t * _R + cchunk - 1) // cchunk
    sup = jax.lax.fori_loop(
        0, nchunks, cross_body, jnp.zeros((1, _R), jnp.float32), unroll=False
    )
    keep0 = jnp.where(sup > _THR, 0.0, 1.0)

    tx1 = x1_ref[:, pl.ds(base, _R)]
    ty1 = y1_ref[:, pl.ds(base, _R)]
    tx2 = x2_ref[:, pl.ds(base, _R)]
    ty2 = y2_ref[:, pl.ds(base, _R)]
    tar = ar_ref[:, pl.ds(base, _R)]
    j = jax.lax.broadcasted_iota(jnp.int32, (1, _R), 1)
    i8 = jax.lax.broadcasted_iota(jnp.int32, (_S, 1), 0)

    def slab_body(s, keep):
        rx1 = sx1_ref[pl.ds(s * _S, _S), :]
        ry1 = sy1_ref[pl.ds(s * _S, _S), :]
        rx2 = sx2_ref[pl.ds(s * _S, _S), :]
        ry2 = sy2_ref[pl.ds(s * _S, _S), :]
        rar = sar_ref[pl.ds(s * _S, _S), :]
        w = jnp.maximum(jnp.minimum(rx2, tx2) - jnp.maximum(rx1, tx1), 0.0)
        h = jnp.maximum(jnp.minimum(ry2, ty2) - jnp.maximum(ry1, ty1), 0.0)
        inter = w * h
        iou8 = inter / (rar + tar - inter + _EPS)

        def sub_body(r, keep):
            k = s * _S + r
            row = jnp.max(jnp.where(i8 == r, iou8, 0.0), axis=0, keepdims=True)
            kk = jnp.max(jnp.where(j == k, keep, 0.0), axis=1, keepdims=True)
            suppress = (row > _THR) & (j > k) & (kk > 0.0)
            return jnp.where(suppress, 0.0, keep)

        return jax.lax.fori_loop(0, _S, sub_body, keep, unroll=True)

    keep = jax.lax.fori_loop(0, _R // _S, slab_body, keep0, unroll=False)

    out_ref[:, pl.ds(base, _R)] = keep
    kept = keep > 0.0
    px1_ref[:, pl.ds(base, _R)] = jnp.where(kept, tx1, _BIG)
    py1_ref[:, pl.ds(base, _R)] = jnp.where(kept, ty1, _BIG)
    px2_ref[:, pl.ds(base, _R)] = jnp.where(kept, tx2, _BIG)
    py2_ref[:, pl.ds(base, _R)] = jnp.where(kept, ty2, _BIG)
    par_ref[:, pl.ds(base, _R)] = jnp.where(kept, tar, 0.0)


def _run_nms(x1, y1, x2, y2, area, interpret=False):
    npad = x1.shape[1]
    cchunk = min(_C, npad)
    spec = pl.BlockSpec((1, npad), lambda t: (0, 0))
    big = pltpu.VMEM((1, npad), jnp.float32)
    small = pltpu.VMEM((_R, 1), jnp.float32)
    keep = pl.pallas_call(
        functools.partial(_nms_kernel, cchunk),
        grid=(npad // _R,),
        in_specs=[spec] * 5,
        out_specs=spec,
        out_shape=jax.ShapeDtypeStruct((1, npad), jnp.float32),
        scratch_shapes=[big] * 5 + [small] * 5,
        interpret=interpret,
    )(x1, y1, x2, y2, area)
    return keep[0]


def kernel(boxes, scores):
    n = scores.shape[0]
    order = jnp.argsort(-scores)
    b = boxes[order]
    blk = max(_R, _C)
    npad = ((n + blk - 1) // blk) * blk
    pad = npad - n
    bp = jnp.concatenate([b, jnp.full((pad, 4), _BIG, jnp.float32)], axis=0)
    x1 = bp[:, 0].reshape(1, npad)
    y1 = bp[:, 1].reshape(1, npad)
    x2 = bp[:, 2].reshape(1, npad)
    y2 = bp[:, 3].reshape(1, npad)
    area = (x2 - x1) * (y2 - y1)
    keep_sorted = _run_nms(x1, y1, x2, y2, area)[:n] > 0.0
    kept = jnp.zeros((n,), dtype=bool).at[order].set(keep_sorted)
    return jnp.where(kept, scores, 0.0)
